# baseline (speedup 1.0000x reference)
import jax
import jax.numpy as jnp
from jax import lax
from jax.experimental import pallas as pl
from jax.experimental.pallas import tpu as pltpu
from jax.experimental.pallas import tpu_sc as plsc

_HIDDEN = 64
_BATCH = 16384
_LMBDA = 0.0001
_NC, _NS, _LANES = 2, 16, 16
_NW = _NC * _NS
_BPW = _BATCH // _NW
_IDXW = 128
_NI = _BPW // _IDXW
_CH = _HIDDEN // _LANES


def _sc_body(h_hbm, t_hbm, r_hbm, ent_hbm, rel_hbm, res_out, sq_out,
             h_v, t_v, r_v, eh_v, et_v, er_v, res_v, sq_v, sem):
    wid = lax.axis_index("s") * _NC + lax.axis_index("c")
    pltpu.sync_copy(h_hbm.at[wid], h_v)
    pltpu.sync_copy(t_hbm.at[wid], t_v)
    pltpu.sync_copy(r_hbm.at[wid], r_v)
    copies = []
    for j in range(_NI):
        sl = pl.ds(j * _IDXW, _IDXW)
        copies.append(pltpu.async_copy(ent_hbm.at[h_v.at[j]], eh_v.at[sl], sem))
        copies.append(pltpu.async_copy(ent_hbm.at[t_v.at[j]], et_v.at[sl], sem))
        copies.append(pltpu.async_copy(rel_hbm.at[r_v.at[j]], er_v.at[sl], sem))
    for cp in copies:
        cp.wait()

    lane_iota = lax.iota(jnp.int32, _LANES)

    def group(g, sq_acc):
        resvec = jnp.zeros((_LANES,), jnp.float32)
        for j in range(_LANES):
            i = g * _LANES + j
            acc = jnp.zeros((_LANES,), jnp.float32)
            for c in range(_CH):
                cs = pl.ds(c * _LANES, _LANES)
                vh = eh_v[i, cs]
                vt = et_v[i, cs]
                vr = er_v[i, cs]
                acc = acc + vh * vt * vr
                sq_acc = sq_acc + (vh * vh + vt * vt + vr * vr)
            resvec = jnp.where(lane_iota == j, jnp.sum(acc), resvec)
        res_v[pl.ds(g * _LANES, _LANES)] = resvec
        return sq_acc

    sq = lax.fori_loop(0, _BPW // _LANES, group,
                       jnp.zeros((_LANES,), jnp.float32))
    sq_v[...] = sq
    pltpu.sync_copy(res_v, res_out.at[pl.ds(wid * _BPW, _BPW)])
    pltpu.sync_copy(sq_v, sq_out.at[wid])


_sc_call = pl.kernel(
    _sc_body,
    out_type=(
        jax.ShapeDtypeStruct((_BATCH,), jnp.float32),
        jax.ShapeDtypeStruct((_NW, _LANES), jnp.float32),
    ),
    mesh=plsc.VectorSubcoreMesh(
        core_axis_name="c", subcore_axis_name="s",
        num_cores=_NC, num_subcores=_NS,
    ),
    scratch_types=[
        pltpu.VMEM((_NI, _IDXW), jnp.int32),
        pltpu.VMEM((_NI, _IDXW), jnp.int32),
        pltpu.VMEM((_NI, _IDXW), jnp.int32),
        pltpu.VMEM((_BPW, _HIDDEN), jnp.float32),
        pltpu.VMEM((_BPW, _HIDDEN), jnp.float32),
        pltpu.VMEM((_BPW, _HIDDEN), jnp.float32),
        pltpu.VMEM((_BPW,), jnp.float32),
        pltpu.VMEM((_LANES,), jnp.float32),
        pltpu.SemaphoreType.DMA,
    ],
    compiler_params=pltpu.CompilerParams(
        needs_layout_passes=False, use_tc_tiling_on_sc=False),
)


def _tc_body(res_ref, y_ref, sq_ref, out_ref):
    z = -(y_ref[...] * res_ref[...])
    sp = jnp.maximum(z, 0.0) + jnp.log1p(jnp.exp(-jnp.abs(z)))
    loss = jnp.sum(sp) / _BATCH
    loss = loss + _LMBDA * (jnp.sum(sq_ref[...]) / (_BATCH * _HIDDEN))
    out_ref[0, 0] = loss


_tc_call = pl.pallas_call(
    _tc_body,
    out_shape=jax.ShapeDtypeStruct((1, 1), jnp.float32),
    out_specs=pl.BlockSpec(memory_space=pltpu.SMEM),
)


def kernel(h, t, r, y, ent_embeddings, rel_embeddings):
    h3 = h.astype(jnp.int32).reshape(_NW, _NI, _IDXW)
    t3 = t.astype(jnp.int32).reshape(_NW, _NI, _IDXW)
    r3 = r.astype(jnp.int32).reshape(_NW, _NI, _IDXW)
    res, sq = _sc_call(h3, t3, r3, ent_embeddings, rel_embeddings)
    loss = _tc_call(res.reshape(128, 128), y.reshape(128, 128), sq)
    return loss[0, 0]


# speedup vs baseline: 1.6573x; 1.6573x over previous
import jax
import jax.numpy as jnp
from jax import lax
from jax.experimental import pallas as pl
from jax.experimental.pallas import tpu as pltpu
from jax.experimental.pallas import tpu_sc as plsc

_HIDDEN = 64
_BATCH = 16384
_LMBDA = 0.0001
_NC, _NS, _LANES = 2, 16, 16
_NW = _NC * _NS
_BPW = _BATCH // _NW
_CH = _HIDDEN // _LANES
_NG = _BPW // _LANES
_PASSR = 128


def _sc_body(hto_hbm, ent_hbm, rel_hbm, res_out, sq_out,
             idx_v, eh_v, et_v, er_v, res_v, sq_v, sem):
    wid = lax.axis_index("s") * _NC + lax.axis_index("c")
    pltpu.sync_copy(hto_hbm.at[pl.ds(wid * 8, 8)], idx_v)

    lane_iota = lax.iota(jnp.int32, _LANES)
    gpp = _PASSR // _LANES

    def make_fire(p):
        def fire(g, carry):
            gs = pl.ds(p * _PASSR + g * _LANES, _LANES)
            hg = idx_v[0, gs]
            tg = idx_v[1, gs]
            rg = idx_v[2, gs]
            for j in range(_LANES):
                row = g * _LANES + j
                pltpu.async_copy(ent_hbm.at[hg[j]], eh_v.at[row], sem)
                pltpu.async_copy(ent_hbm.at[tg[j]], et_v.at[row], sem)
                pltpu.async_copy(rel_hbm.at[rg[j]], er_v.at[row], sem)
            return carry
        return fire

    def make_group(p):
        def group(g, sq_acc):
            resvec = jnp.zeros((_LANES,), jnp.float32)
            for j in range(_LANES):
                i = g * _LANES + j
                acc = jnp.zeros((_LANES,), jnp.float32)
                for c in range(_CH):
                    cs = pl.ds(c * _LANES, _LANES)
                    vh = eh_v[i, cs]
                    vt = et_v[i, cs]
                    vr = er_v[i, cs]
                    acc = acc + vh * vt * vr
                    sq_acc = sq_acc + (vh * vh + vt * vt + vr * vr)
                resvec = jnp.where(lane_iota == j, jnp.sum(acc), resvec)
            res_v[pl.ds(p * _PASSR + g * _LANES, _LANES)] = resvec
            return sq_acc
        return group

    sq = jnp.zeros((_LANES,), jnp.float32)
    for p in range(_BPW // _PASSR):
        lax.fori_loop(0, gpp, make_fire(p), 0)
        pltpu.make_async_copy(ent_hbm.at[pl.ds(0, _PASSR)], eh_v, sem).wait()
        pltpu.make_async_copy(ent_hbm.at[pl.ds(0, _PASSR)], et_v, sem).wait()
        pltpu.make_async_copy(ent_hbm.at[pl.ds(0, _PASSR)], er_v, sem).wait()
        sq = lax.fori_loop(0, gpp, make_group(p), sq)
    sq_v[...] = sq
    pltpu.sync_copy(res_v, res_out.at[pl.ds(wid * _BPW, _BPW)])
    pltpu.sync_copy(sq_v, sq_out.at[pl.ds(wid * _LANES, _LANES)])


_sc_call = pl.kernel(
    _sc_body,
    out_type=(
        jax.ShapeDtypeStruct((_BATCH,), jnp.float32),
        jax.ShapeDtypeStruct((_NW * _LANES,), jnp.float32),
    ),
    mesh=plsc.VectorSubcoreMesh(
        core_axis_name="c", subcore_axis_name="s",
        num_cores=_NC, num_subcores=_NS,
    ),
    scratch_types=[
        pltpu.VMEM((8, _BPW), jnp.int32),
        pltpu.VMEM((_PASSR, _HIDDEN), jnp.float32),
        pltpu.VMEM((_PASSR, _HIDDEN), jnp.float32),
        pltpu.VMEM((_PASSR, _HIDDEN), jnp.float32),
        pltpu.VMEM((_BPW,), jnp.float32),
        pltpu.VMEM((_LANES,), jnp.float32),
        pltpu.SemaphoreType.DMA,
    ],
    compiler_params=pltpu.CompilerParams(
        needs_layout_passes=False, use_tc_tiling_on_sc=True),
)


def _tc_body(res_ref, y_ref, sq_ref, out_ref):
    z = -(y_ref[...] * res_ref[...])
    sp = jnp.maximum(z, 0.0) + jnp.log1p(jnp.exp(-jnp.abs(z)))
    loss = jnp.sum(sp) / _BATCH
    loss = loss + _LMBDA * (jnp.sum(sq_ref[...]) / (_BATCH * _HIDDEN))
    out_ref[0, 0] = loss


_tc_call = pl.pallas_call(
    _tc_body,
    out_shape=jax.ShapeDtypeStruct((1, 1), jnp.float32),
    out_specs=pl.BlockSpec(memory_space=pltpu.SMEM),
)


def kernel(h, t, r, y, ent_embeddings, rel_embeddings):
    idx = jnp.stack(
        [x.astype(jnp.int32).reshape(_NW, _BPW) for x in (h, t, r)], axis=1)
    idx = jnp.pad(idx, ((0, 0), (0, 5), (0, 0))).reshape(_NW * 8, _BPW)
    res, sq = _sc_call(idx, ent_embeddings, rel_embeddings)
    loss = _tc_call(res.reshape(128, 128), y.reshape(128, 128),
                    sq.reshape(_NW, _LANES))
    return loss[0, 0]


# speedup vs baseline: 2.4503x; 1.4785x over previous
import jax
import jax.numpy as jnp
from jax import lax
from jax.experimental import pallas as pl
from jax.experimental.pallas import tpu as pltpu
from jax.experimental.pallas import tpu_sc as plsc

_HIDDEN = 64
_BATCH = 16384
_LMBDA = 0.0001
_NC, _NS, _LANES = 2, 16, 16
_NW = _NC * _NS
_BPW = _BATCH // _NW
_CH = _HIDDEN // _LANES
_PASSR = 128
_PT = _PASSR // 8


def _sc_body(hto_hbm, ent_hbm, rel_hbm, res_out, sq_out,
             idx_v, eh_v, et_v, er_v, res_v, sq_v, sem):
    wid = lax.axis_index("s") * _NC + lax.axis_index("c")
    pltpu.sync_copy(hto_hbm.at[pl.ds(wid * 8, 8)], idx_v)

    lane_iota = lax.iota(jnp.int32, _LANES)
    gpp = _PASSR // _LANES

    def make_fire(p):
        def fire(g, carry):
            gs = pl.ds(p * _PASSR + g * _LANES, _LANES)
            hg = idx_v[0, gs]
            tg = idx_v[1, gs]
            rg = idx_v[2, gs]
            for j in range(_LANES):
                row = g * _LANES + j
                tr, ts = row // 8, row % 8
                pltpu.async_copy(
                    ent_hbm.at[hg[j] >> 3, hg[j] & 7], eh_v.at[tr, ts], sem)
                pltpu.async_copy(
                    ent_hbm.at[tg[j] >> 3, tg[j] & 7], et_v.at[tr, ts], sem)
                pltpu.async_copy(
                    rel_hbm.at[rg[j] >> 3, rg[j] & 7], er_v.at[tr, ts], sem)
            return carry
        return fire

    def make_group(p):
        def group(g, sq_acc):
            resvec = jnp.zeros((_LANES,), jnp.float32)
            for j in range(_LANES):
                i = g * _LANES + j
                ti = i // 8
                si = i % 8
                acc = jnp.zeros((_LANES,), jnp.float32)
                for c in range(_CH):
                    cs = pl.ds(c * _LANES, _LANES)
                    vh = eh_v[ti, si, cs]
                    vt = et_v[ti, si, cs]
                    vr = er_v[ti, si, cs]
                    acc = acc + vh * vt * vr
                    sq_acc = sq_acc + (vh * vh + vt * vt + vr * vr)
                resvec = jnp.where(lane_iota == j, jnp.sum(acc), resvec)
            res_v[pl.ds(p * _PASSR + g * _LANES, _LANES)] = resvec
            return sq_acc
        return group

    sq = jnp.zeros((_LANES,), jnp.float32)
    for p in range(_BPW // _PASSR):
        lax.fori_loop(0, gpp, make_fire(p), 0)
        pltpu.make_async_copy(ent_hbm.at[pl.ds(0, _PT)], eh_v, sem).wait()
        pltpu.make_async_copy(ent_hbm.at[pl.ds(0, _PT)], et_v, sem).wait()
        pltpu.make_async_copy(ent_hbm.at[pl.ds(0, _PT)], er_v, sem).wait()
        sq = lax.fori_loop(0, gpp, make_group(p), sq)
    sq_v[...] = sq
    pltpu.sync_copy(res_v, res_out.at[pl.ds(wid * _BPW, _BPW)])
    pltpu.sync_copy(sq_v, sq_out.at[pl.ds(wid * _LANES, _LANES)])


_sc_call = pl.kernel(
    _sc_body,
    out_type=(
        jax.ShapeDtypeStruct((_BATCH,), jnp.float32),
        jax.ShapeDtypeStruct((_NW * _LANES,), jnp.float32),
    ),
    mesh=plsc.VectorSubcoreMesh(
        core_axis_name="c", subcore_axis_name="s",
        num_cores=_NC, num_subcores=_NS,
    ),
    scratch_types=[
        pltpu.VMEM((8, _BPW), jnp.int32),
        pltpu.VMEM((_PT, 8, _HIDDEN), jnp.float32),
        pltpu.VMEM((_PT, 8, _HIDDEN), jnp.float32),
        pltpu.VMEM((_PT, 8, _HIDDEN), jnp.float32),
        pltpu.VMEM((_BPW,), jnp.float32),
        pltpu.VMEM((_LANES,), jnp.float32),
        pltpu.SemaphoreType.DMA,
    ],
    compiler_params=pltpu.CompilerParams(
        needs_layout_passes=False, use_tc_tiling_on_sc=True),
)


def _tc_body(res_ref, y_ref, sq_ref, out_ref):
    z = -(y_ref[...] * res_ref[...])
    sp = jnp.maximum(z, 0.0) + jnp.log1p(jnp.exp(-jnp.abs(z)))
    loss = jnp.sum(sp) / _BATCH
    loss = loss + _LMBDA * (jnp.sum(sq_ref[...]) / (_BATCH * _HIDDEN))
    out_ref[0, 0] = loss


_tc_call = pl.pallas_call(
    _tc_body,
    out_shape=jax.ShapeDtypeStruct((1, 1), jnp.float32),
    out_specs=pl.BlockSpec(memory_space=pltpu.SMEM),
)


def kernel(h, t, r, y, ent_embeddings, rel_embeddings):
    idx = jnp.stack(
        [x.astype(jnp.int32).reshape(_NW, _BPW) for x in (h, t, r)], axis=1)
    idx = jnp.pad(idx, ((0, 0), (0, 5), (0, 0))).reshape(_NW * 8, _BPW)
    ent3 = ent_embeddings.reshape(-1, 8, _HIDDEN)
    rel3 = rel_embeddings.reshape(-1, 8, _HIDDEN)
    res, sq = _sc_call(idx, ent3, rel3)
    loss = _tc_call(res.reshape(128, 128), y.reshape(128, 128),
                    sq.reshape(_NW, _LANES))
    return loss[0, 0]
